# baseline (speedup 1.0000x reference)
import functools

import jax
import jax.numpy as jnp
from jax import lax
from jax.experimental import pallas as pl
from jax.experimental.pallas import tpu as pltpu
from jax.experimental.pallas import tpu_sc as plsc

N = 10000
E = 320000
H = 128
D_IN = 128
D_EDGE = 16
N_MP = 4

NC = 2
NS = 16
NW = NC * NS
EPW = E // NW
SUB = 80
SLAB = 25
NSLAB = EPW // (SUB * SLAB)
NP = 10240
RPT = NP // NS
ZCH = 128

@functools.cache
def _mesh():
    return plsc.VectorSubcoreMesh(core_axis_name="c", subcore_axis_name="s",
                                  num_cores=NC, num_subcores=NS)


def _elu(x):
    return jnp.where(x > 0, x, jnp.exp(x) - 1.0)


def _ln(h, g, b):
    m = jnp.mean(h, axis=-1, keepdims=True)
    v = jnp.mean((h - m) * (h - m), axis=-1, keepdims=True)
    return (h - m) * lax.rsqrt(v + 1e-5) * g + b


def _mm(a, w):
    return jnp.dot(a, w, preferred_element_type=jnp.float32)


def _enc_body(x, w1, b1, w2, b2, g, b, o):
    h = _elu(_mm(x[...], w1[...]) + b1[...])
    h = _mm(h, w2[...]) + b2[...]
    o[...] = _ln(h, g[...], b[...])


def _prep_body(xh, wa, wb, ao, bo):
    ao[...] = _mm(xh[...], wa[...])
    bo[...] = _mm(xh[...], wb[...])


def _edge_body(gc, gr, eh, w1c, b1, w2, b2, g, b, o):
    t = gc[...] + gr[...] + _mm(eh[...], w1c[...]) + b1[...]
    h = _mm(_elu(t), w2[...]) + b2[...]
    o[...] = eh[...] + _ln(h, g[...], b[...])


def _node_body(xh, p00, p01, p10, p11, d0, d1, wa, wb, c1, w2, c2, g, b, o):
    deg = jnp.maximum(d0[...][:, :1] + d1[...][:, :1], 1.0)
    agg = (p00[...] + p01[...] + p10[...] + p11[...]) / deg
    t = _mm(xh[...], wa[...]) + _mm(agg, wb[...]) + c1[...]
    h = _mm(_elu(t), w2[...]) + c2[...]
    o[...] = xh[...] + _ln(h, g[...], b[...])


def _dec_body(xh, w1, b1, w2, b2, o):
    h = _elu(_mm(xh[...], w1[...]) + b1[...])
    o[...] = _mm(h, w2[...]) + b2[...]


def _row_spec(bm, d):
    return pl.BlockSpec((bm, d), lambda i: (i, 0))


def _rep_spec(shape):
    nd = len(shape)
    return pl.BlockSpec(shape, lambda i: (0,) * nd)


def _tc_call(body, grid, in_specs, out_specs, out_shape):
    return pl.pallas_call(
        body,
        grid=grid,
        in_specs=in_specs,
        out_specs=out_specs,
        out_shape=out_shape,
        compiler_params=pltpu.CompilerParams(
            dimension_semantics=("arbitrary",)),
    )


def _gather_sc(a, bm, col4, row4):
    _, nslab, slab_n, sub = col4.shape
    epw = nslab * slab_n * sub
    ec = NW * epw

    def body(a_hbm, b_hbm, c_hbm, r_hbm, gc_hbm, gr_hbm,
             icb, irb, bufa, bufb, sema, semb):
        cid = lax.axis_index("c")
        sid = lax.axis_index("s")
        wid = sid * NC + cid

        def slab(i, carry):
            base = wid * epw + i * slab_n * sub
            pltpu.sync_copy(c_hbm.at[wid, i], icb)
            pltpu.sync_copy(r_hbm.at[wid, i], irb)
            for k in range(slab_n):
                ca = pltpu.async_copy(a_hbm.at[icb.at[k]], bufa, sema)
                cb = pltpu.async_copy(b_hbm.at[irb.at[k]], bufb, semb)
                ca.wait()
                cb.wait()
                pltpu.sync_copy(bufa, gc_hbm.at[pl.ds(base + k * sub, sub)])
                pltpu.sync_copy(bufb, gr_hbm.at[pl.ds(base + k * sub, sub)])
            return carry

        lax.fori_loop(0, nslab, slab, 0)

    f = pl.kernel(
        body,
        out_type=(jax.ShapeDtypeStruct((ec, H), jnp.float32),
                  jax.ShapeDtypeStruct((ec, H), jnp.float32)),
        mesh=_mesh(),
        scratch_types=[
            pltpu.VMEM((slab_n, sub), jnp.int32),
            pltpu.VMEM((slab_n, sub), jnp.int32),
            pltpu.VMEM((sub, H), jnp.float32),
            pltpu.VMEM((sub, H), jnp.float32),
            pltpu.SemaphoreType.DMA,
            pltpu.SemaphoreType.DMA,
        ],
    )
    return f(a, bm, col4, row4)


def _scatter_sc(src, col4, zer):
    _, nslab, slab_n, sub = col4.shape
    epw = nslab * slab_n * sub

    def body(s_hbm, c_hbm, z_hbm, parts_hbm, acc, ebuf, zbuf, ibuf, obuf):
        cid = lax.axis_index("c")
        sid = lax.axis_index("s")
        wid = sid * NC + cid

        pltpu.sync_copy(z_hbm, zbuf)
        for k in range(RPT // ZCH):
            pltpu.sync_copy(zbuf, acc.at[pl.ds(sid * RPT + k * ZCH, ZCH)])
        plsc.subcore_barrier()

        def slab(i, carry):
            base = wid * epw + i * slab_n * sub
            pltpu.sync_copy(c_hbm.at[wid, i], ibuf)
            for k in range(slab_n):
                pltpu.sync_copy(s_hbm.at[pl.ds(base + k * sub, sub)], ebuf)
                pltpu.sync_copy(ebuf, acc.at[ibuf.at[k]], add=True)
            return carry

        lax.fori_loop(0, nslab, slab, 0)
        plsc.subcore_barrier()

        for k in range(RPT // ZCH):
            r0 = sid * RPT + k * ZCH
            pltpu.sync_copy(acc.at[pl.ds(r0, ZCH)], obuf)
            pltpu.sync_copy(obuf, parts_hbm.at[pl.ds(cid * NP + r0, ZCH)])

    f = pl.kernel(
        body,
        out_type=jax.ShapeDtypeStruct((2 * NP, H), jnp.float32),
        mesh=_mesh(),
        scratch_types=[
            pltpu.VMEM_SHARED((NP, H), jnp.float32),
            pltpu.VMEM((sub, H), jnp.float32),
            pltpu.VMEM((ZCH, H), jnp.float32),
            pltpu.VMEM((slab_n, sub), jnp.int32),
            pltpu.VMEM((ZCH, H), jnp.float32),
        ],
    )
    return f(src, col4, zer)


def _deg_sc(col4, ones, zer):
    def body(c_hbm, o_hbm, z_hbm, parts_hbm, acc, onz, zbuf, ibuf, obuf):
        cid = lax.axis_index("c")
        sid = lax.axis_index("s")
        wid = sid * NC + cid

        pltpu.sync_copy(o_hbm, onz)
        pltpu.sync_copy(z_hbm, zbuf)
        for k in range(RPT // ZCH):
            pltpu.sync_copy(zbuf, acc.at[pl.ds(sid * RPT + k * ZCH, ZCH)])
        plsc.subcore_barrier()

        def slab(i, carry):
            pltpu.sync_copy(c_hbm.at[wid, i], ibuf)
            for k in range(SLAB):
                pltpu.sync_copy(onz, acc.at[ibuf.at[k]], add=True)
            return carry

        lax.fori_loop(0, NSLAB, slab, 0)
        plsc.subcore_barrier()

        for k in range(RPT // ZCH):
            r0 = sid * RPT + k * ZCH
            pltpu.sync_copy(acc.at[pl.ds(r0, ZCH)], obuf)
            pltpu.sync_copy(obuf, parts_hbm.at[pl.ds(cid * NP + r0, ZCH)])

    f = pl.kernel(
        body,
        out_type=jax.ShapeDtypeStruct((2 * NP, H), jnp.float32),
        mesh=_mesh(),
        scratch_types=[
            pltpu.VMEM_SHARED((NP, H), jnp.float32),
            pltpu.VMEM((SUB, H), jnp.float32),
            pltpu.VMEM((ZCH, H), jnp.float32),
            pltpu.VMEM((SLAB, SUB), jnp.int32),
            pltpu.VMEM((ZCH, H), jnp.float32),
        ],
    )
    return f(col4, ones, zer)


BN = 1000
BE = 2000
CH = 2
EC = E // CH
SUBC = 40


def kernel(x, edge_index, edge_attr, pos, batch, params):
    p = params
    row = edge_index[0]
    col = edge_index[1]
    col4 = col.reshape(NW, NSLAB, SLAB, SUB)
    row4 = row.reshape(NW, NSLAB, SLAB, SUB)
    col4s = [col[c * EC:(c + 1) * EC].reshape(NW, -1, SLAB, SUBC)
             for c in range(CH)]
    row4s = [row[c * EC:(c + 1) * EC].reshape(NW, -1, SLAB, SUBC)
             for c in range(CH)]

    def r2(d):
        return d.reshape(1, -1)

    ne = p["node_encode"]
    nn = p["node_encode_norm"]
    xh = _tc_call(
        _enc_body, (N // BN,),
        [_row_spec(BN, D_IN)] + [_rep_spec(s) for s in
                                 [(D_IN, H), (1, H), (H, H), (1, H), (1, H), (1, H)]],
        _row_spec(BN, H), jax.ShapeDtypeStruct((N, H), jnp.float32),
    )(x, ne[0]["W"], r2(ne[0]["b"]), ne[1]["W"], r2(ne[1]["b"]),
      r2(nn["g"]), r2(nn["b"]))

    ee = p["edge_encode"]
    en = p["edge_encode_norm"]
    edge_enc = _tc_call(
        _enc_body, (EC // BE,),
        [_row_spec(BE, D_EDGE)] + [_rep_spec(s) for s in
                                   [(D_EDGE, H), (1, H), (H, H), (1, H), (1, H), (1, H)]],
        _row_spec(BE, H), jax.ShapeDtypeStruct((EC, H), jnp.float32),
    )
    ehs = [edge_enc(edge_attr[c * EC:(c + 1) * EC],
                    ee[0]["W"], r2(ee[0]["b"]), ee[1]["W"], r2(ee[1]["b"]),
                    r2(en["g"]), r2(en["b"])) for c in range(CH)]

    zer = jnp.zeros((ZCH, H), jnp.float32)
    ones = jnp.ones((SUB, H), jnp.float32)
    degparts = _deg_sc(col4, ones, zer)
    d0 = degparts[:N, :16]
    d1 = degparts[NP:NP + N, :16]

    prep = _tc_call(
        _prep_body, (N // BN,),
        [_row_spec(BN, H), _rep_spec((H, H)), _rep_spec((H, H))],
        (_row_spec(BN, H), _row_spec(BN, H)),
        (jax.ShapeDtypeStruct((N, H), jnp.float32),
         jax.ShapeDtypeStruct((N, H), jnp.float32)),
    )

    edge_mlp = _tc_call(
        _edge_body, (EC // BE,),
        [_row_spec(BE, H)] * 3 + [_rep_spec(s) for s in
                                  [(H, H), (1, H), (H, H), (1, H), (1, H), (1, H)]],
        _row_spec(BE, H), jax.ShapeDtypeStruct((EC, H), jnp.float32),
    )

    node_mlp = _tc_call(
        _node_body, (N // BN,),
        [_row_spec(BN, H)] * 5 + [_row_spec(BN, 16)] * 2 +
        [_rep_spec(s) for s in
         [(H, H), (H, H), (1, H), (H, H), (1, H), (1, H), (1, H)]],
        _row_spec(BN, H), jax.ShapeDtypeStruct((N, H), jnp.float32),
    )

    for i in range(N_MP):
        w1 = p["edge_mps"][i][0]["W"]
        b1 = r2(p["edge_mps"][i][0]["b"])
        w2 = p["edge_mps"][i][1]["W"]
        b2 = r2(p["edge_mps"][i][1]["b"])
        eg = r2(p["edge_norms"][i]["g"])
        eb = r2(p["edge_norms"][i]["b"])

        a, bmat = prep(xh, w1[:H], w1[H:2 * H])
        gs = [_gather_sc(a, bmat, col4s[c], row4s[c]) for c in range(CH)]
        partss = []
        for c in range(CH):
            ehs[c] = edge_mlp(gs[c][0], gs[c][1], ehs[c],
                              w1[2 * H:], b1, w2, b2, eg, eb)
            partss.append(_scatter_sc(ehs[c], col4s[c], zer))

        nw1 = p["node_mps"][i][0]["W"]
        nc1 = r2(p["node_mps"][i][0]["b"])
        nw2 = p["node_mps"][i][1]["W"]
        nc2 = r2(p["node_mps"][i][1]["b"])
        ng = r2(p["node_norms"][i]["g"])
        nb = r2(p["node_norms"][i]["b"])
        xh = node_mlp(xh, partss[0][:N], partss[0][NP:NP + N],
                      partss[1][:N], partss[1][NP:NP + N], d0, d1,
                      nw1[:H], nw1[H:], nc1, nw2, nc2, ng, nb)

    nd = p["node_decode"]
    out = _tc_call(
        _dec_body, (N // BN,),
        [_row_spec(BN, H)] + [_rep_spec(s) for s in
                              [(H, H), (1, H), (H, D_IN), (1, D_IN)]],
        _row_spec(BN, D_IN), jax.ShapeDtypeStruct((N, D_IN), jnp.float32),
    )(xh, nd[0]["W"], r2(nd[0]["b"]), nd[1]["W"], r2(nd[1]["b"]))
    return out


# speedup vs baseline: 1.2258x; 1.2258x over previous
import functools

import jax
import jax.numpy as jnp
from jax import lax
from jax.experimental import pallas as pl
from jax.experimental.pallas import tpu as pltpu
from jax.experimental.pallas import tpu_sc as plsc

N = 10000
E = 320000
H = 128
D_IN = 128
D_EDGE = 16
N_MP = 4

NC = 2
NS = 16
NW = NC * NS
EPW = E // NW
SUB = 80
SLAB = 25
NSLAB = EPW // (SUB * SLAB)
NSUB = EPW // SUB
G = 5
NGRP = NSUB // G
GS = 2
NGRPS = NSUB // GS
REMS = NSUB - NGRPS * GS
NP = 10240
RPT = NP // NS
ZCH = 16


@functools.cache
def _mesh():
    return plsc.VectorSubcoreMesh(core_axis_name="c", subcore_axis_name="s",
                                  num_cores=NC, num_subcores=NS)


def _elu(x):
    return jnp.where(x > 0, x, jnp.exp(x) - 1.0)


def _ln(h, g, b):
    m = jnp.mean(h, axis=-1, keepdims=True)
    v = jnp.mean((h - m) * (h - m), axis=-1, keepdims=True)
    return (h - m) * lax.rsqrt(v + 1e-5) * g + b


def _mm(a, w):
    return jnp.dot(a, w, preferred_element_type=jnp.float32)


def _enc_body(x, w1, b1, w2, b2, g, b, o):
    h = _elu(_mm(x[...], w1[...]) + b1[...])
    h = _mm(h, w2[...]) + b2[...]
    o[...] = _ln(h, g[...], b[...])


def _prep_body(xh, wa, wb, ao, bo):
    ao[...] = _mm(xh[...], wa[...])
    bo[...] = _mm(xh[...], wb[...])


def _edge_body(gc, gr, eh, w1c, b1, w2, b2, g, b, o):
    t = gc[...] + gr[...] + _mm(eh[...], w1c[...]) + b1[...]
    h = _mm(_elu(t), w2[...]) + b2[...]
    o[...] = eh[...] + _ln(h, g[...], b[...])


def _node_body(xh, p0, p1, d0, d1, wa, wb, c1, w2, c2, g, b, o):
    deg = jnp.maximum(d0[...][:, :1] + d1[...][:, :1], 1.0)
    agg = (p0[...] + p1[...]) / deg
    t = _mm(xh[...], wa[...]) + _mm(agg, wb[...]) + c1[...]
    h = _mm(_elu(t), w2[...]) + c2[...]
    o[...] = xh[...] + _ln(h, g[...], b[...])


def _dec_body(xh, w1, b1, w2, b2, o):
    h = _elu(_mm(xh[...], w1[...]) + b1[...])
    o[...] = _mm(h, w2[...]) + b2[...]


def _row_spec(bm, d):
    return pl.BlockSpec((bm, d), lambda i: (i, 0))


def _rep_spec(shape):
    nd = len(shape)
    return pl.BlockSpec(shape, lambda i: (0,) * nd)


def _tc_call(body, grid, in_specs, out_specs, out_shape):
    return pl.pallas_call(
        body,
        grid=grid,
        in_specs=in_specs,
        out_specs=out_specs,
        out_shape=out_shape,
        compiler_params=pltpu.CompilerParams(
            dimension_semantics=("arbitrary",)),
    )


def _load_all_idx(c_hbm, wid, icb):
    for i in range(NSLAB):
        pltpu.sync_copy(c_hbm.at[wid, i], icb.at[pl.ds(i * SLAB, SLAB)])


def _gather_sc(a, bm, col4, row4):
    def body(a_hbm, b_hbm, c_hbm, r_hbm, gc_hbm, gr_hbm,
             icb, irb, bufa, bufb, gsem, wsem):
        cid = lax.axis_index("c")
        sid = lax.axis_index("s")
        wid = sid * NC + cid

        def group(gi, carry):
            @pl.when(gi % G == 0)
            def _():
                pltpu.sync_copy(c_hbm.at[wid, gi // G], icb)
                pltpu.sync_copy(r_hbm.at[wid, gi // G], irb)

            k0 = gi * G
            j0 = (gi % G) * G
            for j in range(G):
                pltpu.async_copy(a_hbm.at[icb.at[j0 + j]], bufa.at[j], gsem)
                pltpu.async_copy(b_hbm.at[irb.at[j0 + j]], bufb.at[j], gsem)
            for j in range(G):
                base = wid * EPW + (k0 + j) * SUB
                pltpu.make_async_copy(
                    a_hbm.at[icb.at[j0 + j]], bufa.at[j], gsem).wait()
                pltpu.make_async_copy(
                    b_hbm.at[irb.at[j0 + j]], bufb.at[j], gsem).wait()
                pltpu.async_copy(bufa.at[j], gc_hbm.at[pl.ds(base, SUB)], wsem)
                pltpu.async_copy(bufb.at[j], gr_hbm.at[pl.ds(base, SUB)], wsem)
            for j in range(G):
                base = wid * EPW + (k0 + j) * SUB
                pltpu.make_async_copy(
                    bufa.at[j], gc_hbm.at[pl.ds(base, SUB)], wsem).wait()
                pltpu.make_async_copy(
                    bufb.at[j], gr_hbm.at[pl.ds(base, SUB)], wsem).wait()
            return carry

        lax.fori_loop(0, NGRP, group, 0)

    f = pl.kernel(
        body,
        out_type=(jax.ShapeDtypeStruct((E, H), jnp.float32),
                  jax.ShapeDtypeStruct((E, H), jnp.float32)),
        mesh=_mesh(),
        scratch_types=[
            pltpu.VMEM((SLAB, SUB), jnp.int32),
            pltpu.VMEM((SLAB, SUB), jnp.int32),
            pltpu.VMEM((G, SUB, H), jnp.float32),
            pltpu.VMEM((G, SUB, H), jnp.float32),
            pltpu.SemaphoreType.DMA,
            pltpu.SemaphoreType.DMA,
        ],
    )
    return f(a, bm, col4, row4)


def _scatter_sc(src, col4, zer):
    def body(s_hbm, c_hbm, z_hbm, parts_hbm, acc, ebuf, icb, zo,
             lsem, ssem):
        cid = lax.axis_index("c")
        sid = lax.axis_index("s")
        wid = sid * NC + cid

        pltpu.sync_copy(z_hbm, zo)
        for k in range(RPT // ZCH):
            pltpu.sync_copy(zo, acc.at[pl.ds(sid * RPT + k * ZCH, ZCH)])
        _load_all_idx(c_hbm, wid, icb)
        plsc.subcore_barrier()

        def chunk_fire(k0, j):
            base = wid * EPW + (k0 + j) * SUB
            pltpu.async_copy(s_hbm.at[pl.ds(base, SUB)], ebuf.at[j], lsem)

        def chunk_add(k0, j):
            base = wid * EPW + (k0 + j) * SUB
            pltpu.make_async_copy(
                s_hbm.at[pl.ds(base, SUB)], ebuf.at[j], lsem).wait()
            pltpu.async_copy(ebuf.at[j], acc.at[icb.at[k0 + j]], ssem,
                             add=True)

        def chunk_drain(k0, j):
            pltpu.make_async_copy(
                ebuf.at[j], acc.at[icb.at[k0 + j]], ssem).wait()

        def group(gi, carry):
            k0 = gi * GS
            for j in range(GS):
                chunk_fire(k0, j)
            for j in range(GS):
                chunk_add(k0, j)
            for j in range(GS):
                chunk_drain(k0, j)
            return carry

        lax.fori_loop(0, NGRPS, group, 0)
        k0r = NGRPS * GS
        for j in range(REMS):
            chunk_fire(k0r, j)
        for j in range(REMS):
            chunk_add(k0r, j)
        for j in range(REMS):
            chunk_drain(k0r, j)
        plsc.subcore_barrier()

        for k in range(RPT // ZCH):
            r0 = sid * RPT + k * ZCH
            pltpu.sync_copy(acc.at[pl.ds(r0, ZCH)], zo)
            pltpu.sync_copy(zo, parts_hbm.at[pl.ds(cid * NP + r0, ZCH)])

    f = pl.kernel(
        body,
        out_type=jax.ShapeDtypeStruct((2 * NP, H), jnp.float32),
        mesh=_mesh(),
        scratch_types=[
            pltpu.VMEM_SHARED((NP, H), jnp.float32),
            pltpu.VMEM((GS, SUB, H), jnp.float32),
            pltpu.VMEM((NSUB, SUB), jnp.int32),
            pltpu.VMEM((ZCH, H), jnp.float32),
            pltpu.SemaphoreType.DMA,
            pltpu.SemaphoreType.DMA,
        ],
    )
    return f(src, col4, zer)


def _deg_sc(col4, ones, zer):
    def body(c_hbm, o_hbm, z_hbm, parts_hbm, acc, onz, icb, zo, ssem):
        cid = lax.axis_index("c")
        sid = lax.axis_index("s")
        wid = sid * NC + cid

        pltpu.sync_copy(o_hbm, onz)
        pltpu.sync_copy(z_hbm, zo)
        for k in range(RPT // ZCH):
            pltpu.sync_copy(zo, acc.at[pl.ds(sid * RPT + k * ZCH, ZCH)])
        _load_all_idx(c_hbm, wid, icb)
        plsc.subcore_barrier()

        def fire(k, carry):
            pltpu.async_copy(onz, acc.at[icb.at[k]], ssem, add=True)
            return carry

        def drain(k, carry):
            pltpu.make_async_copy(onz, acc.at[icb.at[k]], ssem).wait()
            return carry

        lax.fori_loop(0, NSUB, fire, 0)
        lax.fori_loop(0, NSUB, drain, 0)
        plsc.subcore_barrier()

        for k in range(RPT // ZCH):
            r0 = sid * RPT + k * ZCH
            pltpu.sync_copy(acc.at[pl.ds(r0, ZCH)], zo)
            pltpu.sync_copy(zo, parts_hbm.at[pl.ds(cid * NP + r0, ZCH)])

    f = pl.kernel(
        body,
        out_type=jax.ShapeDtypeStruct((2 * NP, H), jnp.float32),
        mesh=_mesh(),
        scratch_types=[
            pltpu.VMEM_SHARED((NP, H), jnp.float32),
            pltpu.VMEM((SUB, H), jnp.float32),
            pltpu.VMEM((NSUB, SUB), jnp.int32),
            pltpu.VMEM((ZCH, H), jnp.float32),
            pltpu.SemaphoreType.DMA,
        ],
    )
    return f(col4, ones, zer)


BN = 1000
BE = 2000


def kernel(x, edge_index, edge_attr, pos, batch, params):
    p = params
    row = edge_index[0]
    col = edge_index[1]
    col4 = col.reshape(NW, NSLAB, SLAB, SUB)
    row4 = row.reshape(NW, NSLAB, SLAB, SUB)

    def r2(d):
        return d.reshape(1, -1)

    ne = p["node_encode"]
    nn = p["node_encode_norm"]
    xh = _tc_call(
        _enc_body, (N // BN,),
        [_row_spec(BN, D_IN)] + [_rep_spec(s) for s in
                                 [(D_IN, H), (1, H), (H, H), (1, H), (1, H), (1, H)]],
        _row_spec(BN, H), jax.ShapeDtypeStruct((N, H), jnp.float32),
    )(x, ne[0]["W"], r2(ne[0]["b"]), ne[1]["W"], r2(ne[1]["b"]),
      r2(nn["g"]), r2(nn["b"]))

    ee = p["edge_encode"]
    en = p["edge_encode_norm"]
    eh = _tc_call(
        _enc_body, (E // BE,),
        [_row_spec(BE, D_EDGE)] + [_rep_spec(s) for s in
                                   [(D_EDGE, H), (1, H), (H, H), (1, H), (1, H), (1, H)]],
        _row_spec(BE, H), jax.ShapeDtypeStruct((E, H), jnp.float32),
    )(edge_attr, ee[0]["W"], r2(ee[0]["b"]), ee[1]["W"], r2(ee[1]["b"]),
      r2(en["g"]), r2(en["b"]))

    zer = jnp.zeros((ZCH, H), jnp.float32)
    ones = jnp.ones((SUB, H), jnp.float32)
    degparts = _deg_sc(col4, ones, zer)
    d0 = degparts[:N, :16]
    d1 = degparts[NP:NP + N, :16]

    prep = _tc_call(
        _prep_body, (N // BN,),
        [_row_spec(BN, H), _rep_spec((H, H)), _rep_spec((H, H))],
        (_row_spec(BN, H), _row_spec(BN, H)),
        (jax.ShapeDtypeStruct((N, H), jnp.float32),
         jax.ShapeDtypeStruct((N, H), jnp.float32)),
    )

    edge_mlp = _tc_call(
        _edge_body, (E // BE,),
        [_row_spec(BE, H)] * 3 + [_rep_spec(s) for s in
                                  [(H, H), (1, H), (H, H), (1, H), (1, H), (1, H)]],
        _row_spec(BE, H), jax.ShapeDtypeStruct((E, H), jnp.float32),
    )

    node_mlp = _tc_call(
        _node_body, (N // BN,),
        [_row_spec(BN, H)] * 3 + [_row_spec(BN, 16)] * 2 +
        [_rep_spec(s) for s in
         [(H, H), (H, H), (1, H), (H, H), (1, H), (1, H), (1, H)]],
        _row_spec(BN, H), jax.ShapeDtypeStruct((N, H), jnp.float32),
    )

    for i in range(N_MP):
        w1 = p["edge_mps"][i][0]["W"]
        b1 = r2(p["edge_mps"][i][0]["b"])
        w2 = p["edge_mps"][i][1]["W"]
        b2 = r2(p["edge_mps"][i][1]["b"])
        eg = r2(p["edge_norms"][i]["g"])
        eb = r2(p["edge_norms"][i]["b"])

        a, bmat = prep(xh, w1[:H], w1[H:2 * H])
        gcol, grow = _gather_sc(a, bmat, col4, row4)
        eh = edge_mlp(gcol, grow, eh, w1[2 * H:], b1, w2, b2, eg, eb)

        parts = _scatter_sc(eh, col4, zer)

        nw1 = p["node_mps"][i][0]["W"]
        nc1 = r2(p["node_mps"][i][0]["b"])
        nw2 = p["node_mps"][i][1]["W"]
        nc2 = r2(p["node_mps"][i][1]["b"])
        ng = r2(p["node_norms"][i]["g"])
        nb = r2(p["node_norms"][i]["b"])
        xh = node_mlp(xh, parts[:N], parts[NP:NP + N], d0, d1,
                      nw1[:H], nw1[H:], nc1, nw2, nc2, ng, nb)

    nd = p["node_decode"]
    out = _tc_call(
        _dec_body, (N // BN,),
        [_row_spec(BN, H)] + [_rep_spec(s) for s in
                              [(H, H), (1, H), (H, D_IN), (1, D_IN)]],
        _row_spec(BN, D_IN), jax.ShapeDtypeStruct((N, D_IN), jnp.float32),
    )(xh, nd[0]["W"], r2(nd[0]["b"]), nd[1]["W"], r2(nd[1]["b"]))
    return out
